# baseline (speedup 1.0000x reference)
import jax
import jax.numpy as jnp
from jax.experimental import pallas as pl
from jax.experimental.pallas import tpu as pltpu

_INPUT_DIM = 4096
_NUM_EXPERTS = 64
_TOP_K = 8
_BLOCK_M = 1024
_BLOCK_K = 1024


def _router_kernel(x_ref, wt_ref, gamma_ref, beta_ref, temp_ref,
                   rw_ref, disp_ref, loss_ref,
                   acc, load_acc, z_acc):
    i = pl.program_id(0)
    j = pl.program_id(1)
    nsteps = pl.num_programs(0)
    nk = pl.num_programs(1)

    part = jnp.dot(x_ref[...], wt_ref[pl.ds(j * _BLOCK_K, _BLOCK_K), :],
                   preferred_element_type=jnp.float32)

    @pl.when(j == 0)
    def _first():
        acc[...] = part

    @pl.when(j > 0)
    def _rest():
        acc[...] += part

    @pl.when(j == nk - 1)
    def _epilogue():
        logits = acc[...]

        mu = jnp.mean(logits, axis=-1, keepdims=True)
        var = jnp.mean((logits - mu) ** 2, axis=-1, keepdims=True)
        h = (logits - mu) * jax.lax.rsqrt(var + 1e-5) * gamma_ref[...] + beta_ref[...]
        h = h / (jnp.abs(temp_ref[...]) + 1e-6)

        zsum = jnp.sum(h * h)

        hmax = jnp.max(h, axis=-1, keepdims=True)
        e = jnp.exp(h - hmax)
        w = e / jnp.sum(e, axis=-1, keepdims=True)
        rw_ref[...] = w

        col = jnp.sum(w, axis=0, keepdims=True)

        @pl.when(i == 0)
        def _init():
            load_acc[...] = jnp.zeros_like(load_acc)
            z_acc[0, 0] = 0.0

        load_acc[...] += col
        z_acc[0, 0] += zsum

        lanes = jax.lax.broadcasted_iota(jnp.int32, w.shape, 1)
        wc = w
        disp = jnp.zeros_like(w)
        ksum = jnp.zeros((w.shape[0], 1), jnp.float32)
        for _ in range(_TOP_K):
            m = jnp.max(wc, axis=-1, keepdims=True)
            ismax = wc == m
            first = jnp.min(jnp.where(ismax, lanes, _NUM_EXPERTS), axis=-1,
                            keepdims=True)
            chosen = lanes == first
            disp = jnp.where(chosen, w, disp)
            ksum = ksum + m
            wc = jnp.where(chosen, -jnp.inf, wc)
        disp_ref[...] = disp / (ksum + 1e-6)

        @pl.when(i == nsteps - 1)
        def _finalize():
            n_rows = nsteps * _BLOCK_M
            actual = load_acc[...] / n_rows
            ideal = 1.0 / _NUM_EXPERTS
            lb = jnp.sum(ideal * (jnp.log(ideal) - jnp.log(actual))) / _NUM_EXPERTS
            z = z_acc[0, 0] / (n_rows * _NUM_EXPERTS)
            loss_ref[...] = jnp.reshape(0.01 * z + 0.01 * lb, (1, 1))


@jax.jit
def kernel(x, W, gamma, beta, temperature):
    B, S, D = x.shape
    flat = x.reshape(-1, D)
    N = flat.shape[0]
    wt = W.T
    grid = (N // _BLOCK_M, D // _BLOCK_K)

    rw, disp, loss = pl.pallas_call(
        _router_kernel,
        grid=grid,
        in_specs=[
            pl.BlockSpec((_BLOCK_M, _BLOCK_K), lambda i, j: (i, j)),
            pl.BlockSpec((D, _NUM_EXPERTS), lambda i, j: (0, 0)),
            pl.BlockSpec((1, _NUM_EXPERTS), lambda i, j: (0, 0)),
            pl.BlockSpec((1, _NUM_EXPERTS), lambda i, j: (0, 0)),
            pl.BlockSpec((1, 1), lambda i, j: (0, 0)),
        ],
        out_specs=[
            pl.BlockSpec((_BLOCK_M, _NUM_EXPERTS), lambda i, j: (i, 0)),
            pl.BlockSpec((_BLOCK_M, _NUM_EXPERTS), lambda i, j: (i, 0)),
            pl.BlockSpec((1, 1), lambda i, j: (0, 0)),
        ],
        out_shape=[
            jax.ShapeDtypeStruct((N, _NUM_EXPERTS), jnp.float32),
            jax.ShapeDtypeStruct((N, _NUM_EXPERTS), jnp.float32),
            jax.ShapeDtypeStruct((1, 1), jnp.float32),
        ],
        scratch_shapes=[
            pltpu.VMEM((_BLOCK_M, _NUM_EXPERTS), jnp.float32),
            pltpu.VMEM((1, _NUM_EXPERTS), jnp.float32),
            pltpu.SMEM((1, 1), jnp.float32),
        ],
    )(flat, wt, gamma.reshape(1, -1), beta.reshape(1, -1),
      temperature.reshape(1, 1))

    return (jax.lax.stop_gradient(rw),
            disp.reshape(B, S, _NUM_EXPERTS),
            loss[0, 0])


# speedup vs baseline: 1.0833x; 1.0833x over previous
import functools

import jax
import jax.numpy as jnp
from jax import lax
from jax.experimental import pallas as pl
from jax.experimental.pallas import tpu as pltpu
from jax.experimental.pallas import tpu_sc as plsc

_INPUT_DIM = 4096
_NUM_EXPERTS = 64
_TOP_K = 8
_BLOCK_M = 1024
_N_ROWS = 8192
_NC = 2
_NS = 16
_NW = _NC * _NS
_ROWS_PER_W = _N_ROWS // _NW


def _router_tc_kernel(x_ref, wt_ref, gamma_ref, beta_ref, temp_ref,
                      rw_ref, loss_ref, load_acc, z_acc):
    i = pl.program_id(0)
    nsteps = pl.num_programs(0)

    logits = jnp.dot(x_ref[...], wt_ref[...], preferred_element_type=jnp.float32)

    mu = jnp.mean(logits, axis=-1, keepdims=True)
    var = jnp.mean((logits - mu) ** 2, axis=-1, keepdims=True)
    h = (logits - mu) * jax.lax.rsqrt(var + 1e-5) * gamma_ref[...] + beta_ref[...]
    h = h / (jnp.abs(temp_ref[...]) + 1e-6)

    zsum = jnp.sum(h * h)

    hmax = jnp.max(h, axis=-1, keepdims=True)
    e = jnp.exp(h - hmax)
    w = e / jnp.sum(e, axis=-1, keepdims=True)
    rw_ref[...] = w

    col = jnp.sum(w, axis=0, keepdims=True)

    @pl.when(i == 0)
    def _init():
        load_acc[...] = jnp.zeros_like(load_acc)
        z_acc[0, 0] = 0.0

    load_acc[...] += col
    z_acc[0, 0] += zsum

    @pl.when(i == nsteps - 1)
    def _finalize():
        n_rows = nsteps * _BLOCK_M
        actual = load_acc[...] / n_rows
        ideal = 1.0 / _NUM_EXPERTS
        lb = jnp.sum(ideal * (jnp.log(ideal) - jnp.log(actual))) / _NUM_EXPERTS
        z = z_acc[0, 0] / (n_rows * _NUM_EXPERTS)
        loss_ref[...] = jnp.reshape(0.01 * z + 0.01 * lb, (1, 1))


_GDN = lax.GatherDimensionNumbers(offset_dims=(), collapsed_slice_dims=(0,),
                                  start_index_map=(0,))


def _bcast_lane0(v):
    idx = jnp.zeros((16, 1), jnp.int32)
    return lax.gather(v, idx, _GDN, (1,),
                      mode=lax.GatherScatterMode.PROMISE_IN_BOUNDS)


def _sc_topk_kernel(rw_hbm, disp_hbm, rw_v, disp_v):
    wid = lax.axis_index("s") * _NC + lax.axis_index("c")
    base = wid * _ROWS_PER_W
    pltpu.sync_copy(rw_hbm.at[pl.ds(base, _ROWS_PER_W)], rw_v)

    lane = jnp.arange(16, dtype=jnp.int32)
    iotas = [lane + 16 * c for c in range(4)]

    def row_body(r, carry):
        wc = [rw_v[r, pl.ds(16 * c, 16)] for c in range(4)]
        dacc = [jnp.zeros((16,), jnp.float32) for _ in range(4)]
        ksum = jnp.zeros((16,), jnp.float32)
        for _ in range(_TOP_K):
            m4 = jnp.maximum(jnp.maximum(wc[0], wc[1]),
                             jnp.maximum(wc[2], wc[3]))
            srt, _unused = plsc.sort_key_val(m4, lane, descending=True)
            mxb = _bcast_lane0(srt)
            cand = [jnp.where(wc[c] == mxb, iotas[c], _NUM_EXPERTS)
                    for c in range(4)]
            imin4 = jnp.minimum(jnp.minimum(cand[0], cand[1]),
                                jnp.minimum(cand[2], cand[3]))
            isrt, _unused2 = plsc.sort_key_val(imin4, lane, descending=False)
            iminb = _bcast_lane0(isrt)
            chosen = [iotas[c] == iminb for c in range(4)]
            dacc = [jnp.where(chosen[c], mxb, dacc[c]) for c in range(4)]
            wc = [jnp.where(chosen[c], jnp.float32(-1.0), wc[c])
                  for c in range(4)]
            ksum = ksum + mxb
        scale = 1.0 / (ksum + 1e-6)
        for c in range(4):
            disp_v[r, pl.ds(16 * c, 16)] = dacc[c] * scale
        return carry

    lax.fori_loop(0, _ROWS_PER_W, row_body, 0)
    pltpu.sync_copy(disp_v, disp_hbm.at[pl.ds(base, _ROWS_PER_W)])


def _make_sc_topk():
    return pl.kernel(
        _sc_topk_kernel,
        out_type=jax.ShapeDtypeStruct((_N_ROWS, _NUM_EXPERTS), jnp.float32),
        mesh=plsc.VectorSubcoreMesh(core_axis_name="c", subcore_axis_name="s",
                                    num_cores=_NC, num_subcores=_NS),
        scratch_types=[
            pltpu.VMEM((_ROWS_PER_W, _NUM_EXPERTS), jnp.float32),
            pltpu.VMEM((_ROWS_PER_W, _NUM_EXPERTS), jnp.float32),
        ],
        compiler_params=pltpu.CompilerParams(needs_layout_passes=False),
    )


@jax.jit
def kernel(x, W, gamma, beta, temperature):
    B, S, D = x.shape
    flat = x.reshape(-1, D)
    N = flat.shape[0]
    wt = W.T
    grid = N // _BLOCK_M

    rw, loss = pl.pallas_call(
        _router_tc_kernel,
        grid=(grid,),
        in_specs=[
            pl.BlockSpec((_BLOCK_M, D), lambda i: (i, 0)),
            pl.BlockSpec((D, _NUM_EXPERTS), lambda i: (0, 0)),
            pl.BlockSpec((1, _NUM_EXPERTS), lambda i: (0, 0)),
            pl.BlockSpec((1, _NUM_EXPERTS), lambda i: (0, 0)),
            pl.BlockSpec((1, 1), lambda i: (0, 0)),
        ],
        out_specs=[
            pl.BlockSpec((_BLOCK_M, _NUM_EXPERTS), lambda i: (i, 0)),
            pl.BlockSpec((1, 1), lambda i: (0, 0)),
        ],
        out_shape=[
            jax.ShapeDtypeStruct((N, _NUM_EXPERTS), jnp.float32),
            jax.ShapeDtypeStruct((1, 1), jnp.float32),
        ],
        scratch_shapes=[
            pltpu.VMEM((1, _NUM_EXPERTS), jnp.float32),
            pltpu.SMEM((1, 1), jnp.float32),
        ],
    )(flat, wt, gamma.reshape(1, -1), beta.reshape(1, -1),
      temperature.reshape(1, 1))

    disp = _make_sc_topk()(rw)

    return (jax.lax.stop_gradient(rw),
            disp.reshape(B, S, _NUM_EXPERTS),
            loss[0, 0])


# speedup vs baseline: 1.4225x; 1.3131x over previous
import jax
import jax.numpy as jnp
from jax.experimental import pallas as pl
from jax.experimental.pallas import tpu as pltpu

_INPUT_DIM = 4096
_NUM_EXPERTS = 64
_TOP_K = 8
_BLOCK_M = 1024


def _router_kernel(x_ref, wt_ref, gamma_ref, beta_ref, temp_ref,
                   rw_ref, disp_ref, loss_ref,
                   load_acc, z_acc):
    i = pl.program_id(0)
    nsteps = pl.num_programs(0)

    logits = jnp.dot(x_ref[...], wt_ref[...], preferred_element_type=jnp.float32)

    mu = jnp.mean(logits, axis=-1, keepdims=True)
    var = jnp.mean((logits - mu) ** 2, axis=-1, keepdims=True)
    h = (logits - mu) * jax.lax.rsqrt(var + 1e-5) * gamma_ref[...] + beta_ref[...]
    h = h / (jnp.abs(temp_ref[...]) + 1e-6)

    zsum = jnp.sum(h * h)

    hmax = jnp.max(h, axis=-1, keepdims=True)
    e = jnp.exp(h - hmax)
    w = e / jnp.sum(e, axis=-1, keepdims=True)
    rw_ref[...] = w

    col = jnp.sum(w, axis=0, keepdims=True)

    @pl.when(i == 0)
    def _init():
        load_acc[...] = jnp.zeros_like(load_acc)
        z_acc[0, 0] = 0.0

    load_acc[...] += col
    z_acc[0, 0] += zsum

    lanes = jax.lax.broadcasted_iota(jnp.int32, w.shape, 1)
    wc = w
    disp = jnp.zeros_like(w)
    ksum = jnp.zeros((w.shape[0], 1), jnp.float32)
    for _ in range(_TOP_K):
        m = jnp.max(wc, axis=-1, keepdims=True)
        ismax = wc == m
        first = jnp.min(jnp.where(ismax, lanes, _NUM_EXPERTS), axis=-1,
                        keepdims=True)
        chosen = lanes == first
        disp = jnp.where(chosen, w, disp)
        ksum = ksum + m
        wc = jnp.where(chosen, -jnp.inf, wc)
    disp_ref[...] = disp / (ksum + 1e-6)

    @pl.when(i == nsteps - 1)
    def _finalize():
        n_rows = nsteps * _BLOCK_M
        actual = load_acc[...] / n_rows
        ideal = 1.0 / _NUM_EXPERTS
        lb = jnp.sum(ideal * (jnp.log(ideal) - jnp.log(actual))) / _NUM_EXPERTS
        z = z_acc[0, 0] / (n_rows * _NUM_EXPERTS)
        loss_ref[...] = jnp.reshape(0.01 * z + 0.01 * lb, (1, 1))


@jax.jit
def kernel(x, W, gamma, beta, temperature):
    B, S, D = x.shape
    flat = x.reshape(-1, D)
    N = flat.shape[0]
    wt = W.T
    grid = N // _BLOCK_M

    rw, disp, loss = pl.pallas_call(
        _router_kernel,
        grid=(grid,),
        in_specs=[
            pl.BlockSpec((_BLOCK_M, D), lambda i: (i, 0)),
            pl.BlockSpec((D, _NUM_EXPERTS), lambda i: (0, 0)),
            pl.BlockSpec((1, _NUM_EXPERTS), lambda i: (0, 0)),
            pl.BlockSpec((1, _NUM_EXPERTS), lambda i: (0, 0)),
            pl.BlockSpec((1, 1), lambda i: (0, 0)),
        ],
        out_specs=[
            pl.BlockSpec((_BLOCK_M, _NUM_EXPERTS), lambda i: (i, 0)),
            pl.BlockSpec((_BLOCK_M, _NUM_EXPERTS), lambda i: (i, 0)),
            pl.BlockSpec((1, 1), lambda i: (0, 0)),
        ],
        out_shape=[
            jax.ShapeDtypeStruct((N, _NUM_EXPERTS), jnp.float32),
            jax.ShapeDtypeStruct((N, _NUM_EXPERTS), jnp.float32),
            jax.ShapeDtypeStruct((1, 1), jnp.float32),
        ],
        scratch_shapes=[
            pltpu.VMEM((1, _NUM_EXPERTS), jnp.float32),
            pltpu.SMEM((1, 1), jnp.float32),
        ],
    )(flat, wt, gamma.reshape(1, -1), beta.reshape(1, -1),
      temperature.reshape(1, 1))

    return (jax.lax.stop_gradient(rw),
            disp.reshape(B, S, _NUM_EXPERTS),
            loss[0, 0])
